# baseline (speedup 1.0000x reference)
import jax
import jax.numpy as jnp
from jax.experimental import pallas as pl
from jax.experimental.pallas import tpu as pltpu

EMB = 512
HID = 1024


def _xproj_body(x_ref, w_ref, b_ref, o_ref):
    o_ref[...] = (
        jnp.dot(x_ref[...], w_ref[...], preferred_element_type=jnp.float32)
        + b_ref[...]
    )


def _lstm_body(px_ref, wc_ref, h_ref, cfin_ref, hfin_ref):
    t = pl.program_id(0)

    @pl.when(t == 0)
    def _init():
        cfin_ref[...] = jnp.zeros_like(cfin_ref)

    c = cfin_ref[...]
    g = px_ref[...] + jnp.dot(c, wc_ref[...], preferred_element_type=jnp.float32)
    f = jax.nn.sigmoid(g[:, :HID])
    i = jax.nn.sigmoid(g[:, HID:2 * HID])
    gc = jnp.tanh(g[:, 2 * HID:3 * HID])
    o = jax.nn.sigmoid(g[:, 3 * HID:])
    c_new = f * c + i * gc
    cfin_ref[...] = c_new
    h = jnp.tanh(c_new) * o
    h_ref[...] = h

    @pl.when(t == pl.num_programs(0) - 1)
    def _fin():
        hfin_ref[...] = h


def kernel(x_emb, W_F, b_F, W_I, b_I, W_C, b_C, W_O, b_O):
    B, T, _ = x_emb.shape
    H4 = 4 * HID

    W_x = jnp.concatenate([W_F[:EMB], W_I[:EMB], W_C[:EMB], W_O[:EMB]], axis=1)
    W_c = jnp.concatenate([W_F[EMB:], W_I[EMB:], W_C[EMB:], W_O[EMB:]], axis=1)
    b = jnp.concatenate([b_F, b_I, b_C, b_O])[None, :]

    x2d = x_emb.reshape(B * T, EMB)
    bm = 256
    px = pl.pallas_call(
        _xproj_body,
        out_shape=jax.ShapeDtypeStruct((B * T, H4), jnp.float32),
        grid=(B * T // bm,),
        in_specs=[
            pl.BlockSpec((bm, EMB), lambda i: (i, 0)),
            pl.BlockSpec((EMB, H4), lambda i: (0, 0)),
            pl.BlockSpec((1, H4), lambda i: (0, 0)),
        ],
        out_specs=pl.BlockSpec((bm, H4), lambda i: (i, 0)),
        compiler_params=pltpu.CompilerParams(
            dimension_semantics=("parallel",),
            vmem_limit_bytes=48 * 1024 * 1024,
        ),
        name="lstm_xproj",
    )(x2d, W_x, b)

    px_bt = px.reshape(B, T * H4)
    h_bt, c_fin, h_fin = pl.pallas_call(
        _lstm_body,
        out_shape=[
            jax.ShapeDtypeStruct((B, T * HID), jnp.float32),
            jax.ShapeDtypeStruct((B, HID), jnp.float32),
            jax.ShapeDtypeStruct((B, HID), jnp.float32),
        ],
        grid=(T,),
        in_specs=[
            pl.BlockSpec((B, H4), lambda t: (0, t)),
            pl.BlockSpec((HID, H4), lambda t: (0, 0)),
        ],
        out_specs=[
            pl.BlockSpec((B, HID), lambda t: (0, t)),
            pl.BlockSpec((B, HID), lambda t: (0, 0)),
            pl.BlockSpec((B, HID), lambda t: (0, 0)),
        ],
        compiler_params=pltpu.CompilerParams(
            dimension_semantics=("arbitrary",),
            vmem_limit_bytes=48 * 1024 * 1024,
        ),
        name="lstm_recurrent",
    )(px_bt, W_c)

    all_h = h_bt.reshape(B, T, HID)
    return all_h, c_fin, h_fin


# speedup vs baseline: 10.1407x; 10.1407x over previous
import jax
import jax.numpy as jnp
from jax.experimental import pallas as pl
from jax.experimental.pallas import tpu as pltpu

EMB = 512
HID = 1024


def _xproj_body(x_ref, w_ref, b_ref, o_ref):
    o_ref[...] = (
        jnp.dot(x_ref[...], w_ref[...], preferred_element_type=jnp.float32)
        + b_ref[...]
    )


def _lstm_body(px_ref, wc_ref, h_ref, cfin_ref, hfin_ref):
    t = pl.program_id(0)

    @pl.when(t == 0)
    def _init():
        cfin_ref[...] = jnp.zeros_like(cfin_ref)

    c = cfin_ref[...]
    g = px_ref[0] + jnp.dot(c, wc_ref[...], preferred_element_type=jnp.float32)
    f = jax.nn.sigmoid(g[:, :HID])
    i = jax.nn.sigmoid(g[:, HID:2 * HID])
    gc = jnp.tanh(g[:, 2 * HID:3 * HID])
    o = jax.nn.sigmoid(g[:, 3 * HID:])
    c_new = f * c + i * gc
    cfin_ref[...] = c_new
    h = jnp.tanh(c_new) * o
    h_ref[0] = h

    @pl.when(t == pl.num_programs(0) - 1)
    def _fin():
        hfin_ref[...] = h


def kernel(x_emb, W_F, b_F, W_I, b_I, W_C, b_C, W_O, b_O):
    B, T, _ = x_emb.shape
    H4 = 4 * HID

    W_x = jnp.concatenate([W_F[:EMB], W_I[:EMB], W_C[:EMB], W_O[:EMB]], axis=1)
    W_c = jnp.concatenate([W_F[EMB:], W_I[EMB:], W_C[EMB:], W_O[EMB:]], axis=1)
    b = jnp.concatenate([b_F, b_I, b_C, b_O])[None, :]

    x2d = jnp.swapaxes(x_emb, 0, 1).reshape(T * B, EMB)
    bm = 256
    px = pl.pallas_call(
        _xproj_body,
        out_shape=jax.ShapeDtypeStruct((T * B, H4), jnp.float32),
        grid=(T * B // bm,),
        in_specs=[
            pl.BlockSpec((bm, EMB), lambda i: (i, 0)),
            pl.BlockSpec((EMB, H4), lambda i: (0, 0)),
            pl.BlockSpec((1, H4), lambda i: (0, 0)),
        ],
        out_specs=pl.BlockSpec((bm, H4), lambda i: (i, 0)),
        compiler_params=pltpu.CompilerParams(
            dimension_semantics=("parallel",),
            vmem_limit_bytes=48 * 1024 * 1024,
        ),
        name="lstm_xproj",
    )(x2d, W_x, b)

    px3 = px.reshape(T, B, H4)
    h_tbh, c_fin, h_fin = pl.pallas_call(
        _lstm_body,
        out_shape=[
            jax.ShapeDtypeStruct((T, B, HID), jnp.float32),
            jax.ShapeDtypeStruct((B, HID), jnp.float32),
            jax.ShapeDtypeStruct((B, HID), jnp.float32),
        ],
        grid=(T,),
        in_specs=[
            pl.BlockSpec((1, B, H4), lambda t: (t, 0, 0)),
            pl.BlockSpec((HID, H4), lambda t: (0, 0)),
        ],
        out_specs=[
            pl.BlockSpec((1, B, HID), lambda t: (t, 0, 0)),
            pl.BlockSpec((B, HID), lambda t: (0, 0)),
            pl.BlockSpec((B, HID), lambda t: (0, 0)),
        ],
        compiler_params=pltpu.CompilerParams(
            dimension_semantics=("arbitrary",),
            vmem_limit_bytes=48 * 1024 * 1024,
        ),
        name="lstm_recurrent",
    )(px3, W_c)

    all_h = jnp.swapaxes(h_tbh, 0, 1)
    return all_h, c_fin, h_fin


# speedup vs baseline: 10.2040x; 1.0062x over previous
import jax
import jax.numpy as jnp
from jax.experimental import pallas as pl
from jax.experimental.pallas import tpu as pltpu

EMB = 512
HID = 1024


def _xproj_body(x_ref, w_ref, b_ref, o_ref):
    o_ref[...] = (
        jnp.dot(x_ref[...], w_ref[...], preferred_element_type=jnp.float32)
        + b_ref[...]
    )


def _lstm_body(px_ref, wc_ref, h_ref, cfin_ref, hfin_ref):
    t = pl.program_id(0)

    @pl.when(t == 0)
    def _init():
        cfin_ref[...] = jnp.zeros_like(cfin_ref)

    c = cfin_ref[...]
    g = px_ref[0] + jax.lax.dot_general(
        c, wc_ref[...], (((1,), (0,)), ((), ())),
        preferred_element_type=jnp.float32)
    f = jax.nn.sigmoid(g[:, :HID])
    i = jax.nn.sigmoid(g[:, HID:2 * HID])
    gc = jnp.tanh(g[:, 2 * HID:3 * HID])
    o = jax.nn.sigmoid(g[:, 3 * HID:])
    c_new = f * c + i * gc
    cfin_ref[...] = c_new
    h = jnp.tanh(c_new) * o
    h_ref[0] = h

    @pl.when(t == pl.num_programs(0) - 1)
    def _fin():
        hfin_ref[...] = h


def kernel(x_emb, W_F, b_F, W_I, b_I, W_C, b_C, W_O, b_O):
    B, T, _ = x_emb.shape
    H4 = 4 * HID

    W_x = jnp.concatenate([W_F[:EMB], W_I[:EMB], W_C[:EMB], W_O[:EMB]], axis=1)
    W_c = jnp.concatenate(
        [W_F[EMB:], W_I[EMB:], W_C[EMB:], W_O[EMB:]], axis=1
    ).astype(jnp.bfloat16)
    b = jnp.concatenate([b_F, b_I, b_C, b_O])[None, :]

    x2d = jnp.swapaxes(x_emb, 0, 1).reshape(T * B, EMB)
    bm = 256
    px = pl.pallas_call(
        _xproj_body,
        out_shape=jax.ShapeDtypeStruct((T * B, H4), jnp.float32),
        grid=(T * B // bm,),
        in_specs=[
            pl.BlockSpec((bm, EMB), lambda i: (i, 0)),
            pl.BlockSpec((EMB, H4), lambda i: (0, 0)),
            pl.BlockSpec((1, H4), lambda i: (0, 0)),
        ],
        out_specs=pl.BlockSpec((bm, H4), lambda i: (i, 0)),
        compiler_params=pltpu.CompilerParams(
            dimension_semantics=("parallel",),
            vmem_limit_bytes=48 * 1024 * 1024,
        ),
        name="lstm_xproj",
    )(x2d, W_x, b)

    px3 = px.reshape(T, B, H4)
    h_tbh, c_fin, h_fin = pl.pallas_call(
        _lstm_body,
        out_shape=[
            jax.ShapeDtypeStruct((T, B, HID), jnp.float32),
            jax.ShapeDtypeStruct((B, HID), jnp.float32),
            jax.ShapeDtypeStruct((B, HID), jnp.float32),
        ],
        grid=(T,),
        in_specs=[
            pl.BlockSpec((1, B, H4), lambda t: (t, 0, 0)),
            pl.BlockSpec((HID, H4), lambda t: (0, 0)),
        ],
        out_specs=[
            pl.BlockSpec((1, B, HID), lambda t: (t, 0, 0)),
            pl.BlockSpec((B, HID), lambda t: (0, 0)),
            pl.BlockSpec((B, HID), lambda t: (0, 0)),
        ],
        compiler_params=pltpu.CompilerParams(
            dimension_semantics=("arbitrary",),
            vmem_limit_bytes=48 * 1024 * 1024,
        ),
        name="lstm_recurrent",
    )(px3, W_c)

    all_h = jnp.swapaxes(h_tbh, 0, 1)
    return all_h, c_fin, h_fin
